# baseline (speedup 1.0000x reference)
import jax
import jax.numpy as jnp
from jax.experimental import pallas as pl
from jax.experimental.pallas import tpu as pltpu


_BLOCK_ROWS = 20000


def _node_table_kernel(x_ref, emb_ref, w_ref, b_ref, out_ref):
    x = x_ref[...]
    proj = jax.lax.dot_general(
        x, w_ref[...],
        dimension_numbers=(((1,), (1,)), ((), ())),
        preferred_element_type=jnp.float32,
    )
    out_ref[...] = proj + b_ref[...] + emb_ref[...]


def kernel(node_features, emb_weight, proj_w, proj_b):
    n, d = node_features.shape
    grid = -(-n // _BLOCK_ROWS)
    b2d = proj_b.reshape(1, -1)
    return pl.pallas_call(
        _node_table_kernel,
        grid=(grid,),
        in_specs=[
            pl.BlockSpec((_BLOCK_ROWS, d), lambda i: (i, 0)),
            pl.BlockSpec((_BLOCK_ROWS, d), lambda i: (i, 0)),
            pl.BlockSpec(proj_w.shape, lambda i: (0, 0)),
            pl.BlockSpec((1, d), lambda i: (0, 0)),
        ],
        out_specs=pl.BlockSpec((_BLOCK_ROWS, d), lambda i: (i, 0)),
        out_shape=jax.ShapeDtypeStruct((n, d), jnp.float32),
        compiler_params=pltpu.CompilerParams(
            dimension_semantics=("parallel",),
            vmem_limit_bytes=67_000_000,
        ),
    )(node_features, emb_weight, proj_w, b2d)


# speedup vs baseline: 1.0219x; 1.0219x over previous
import jax
import jax.numpy as jnp
from jax.experimental import pallas as pl
from jax.experimental.pallas import tpu as pltpu


_BLOCK_ROWS = 10000


def _node_table_kernel(x_ref, emb_ref, w_ref, b_ref, out_ref):
    x = x_ref[...]
    proj = jax.lax.dot_general(
        x, w_ref[...],
        dimension_numbers=(((1,), (1,)), ((), ())),
        preferred_element_type=jnp.float32,
    )
    out_ref[...] = proj + b_ref[...] + emb_ref[...]


def kernel(node_features, emb_weight, proj_w, proj_b):
    n, d = node_features.shape
    grid = -(-n // _BLOCK_ROWS)
    b2d = proj_b.reshape(1, -1)
    return pl.pallas_call(
        _node_table_kernel,
        grid=(grid,),
        in_specs=[
            pl.BlockSpec((_BLOCK_ROWS, d), lambda i: (i, 0)),
            pl.BlockSpec((_BLOCK_ROWS, d), lambda i: (i, 0)),
            pl.BlockSpec(proj_w.shape, lambda i: (0, 0)),
            pl.BlockSpec((1, d), lambda i: (0, 0)),
        ],
        out_specs=pl.BlockSpec((_BLOCK_ROWS, d), lambda i: (i, 0)),
        out_shape=jax.ShapeDtypeStruct((n, d), jnp.float32),
        compiler_params=pltpu.CompilerParams(
            dimension_semantics=("parallel",),
            vmem_limit_bytes=67_000_000,
        ),
    )(node_features, emb_weight, proj_w, b2d)
